# Initial kernel scaffold; baseline (speedup 1.0000x reference)
import functools

import jax
import jax.numpy as jnp
from jax import lax
from jax.experimental import pallas as pl
from jax.experimental.pallas import tpu as pltpu
from jax.experimental.pallas import tpu_sc as plsc

HID = 512
NG = 512
GPAD = 544
N = 10000
NC, NS, L = 2, 16, 16
NW = NC * NS
NPAD = 10240
CHUNK = NPAD // NW
SUB = 64
STEPS = 6

_mesh = plsc.VectorSubcoreMesh(core_axis_name="c", subcore_axis_name="s")


@functools.partial(
    pl.kernel,
    out_type=(
        jax.ShapeDtypeStruct((NPAD,), jnp.float32),
        jax.ShapeDtypeStruct((NW, GPAD), jnp.float32),
    ),
    mesh=_mesh,
    scratch_types=[
        pltpu.VMEM((CHUNK,), jnp.int32),
        pltpu.VMEM((SUB, HID), jnp.float32),
        pltpu.VMEM((SUB, HID), jnp.float32),
        pltpu.VMEM((CHUNK,), jnp.float32),
        pltpu.VMEM((GPAD,), jnp.float32),
        pltpu.SemaphoreType.DMA,
    ],
)
def _p1(x_hbm, idx_hbm, q_hbm, e_hbm, mpart_hbm, idx_v, x_v, q_v, e_v, m_v, sem):
    wid = lax.axis_index("s") * NC + lax.axis_index("c")
    base = wid * CHUNK
    neg = jnp.full((L,), -jnp.inf, jnp.float32)
    for v in range(GPAD // L):
        m_v[pl.ds(v * L, L)] = neg
    pltpu.sync_copy(idx_hbm.at[pl.ds(base, CHUNK)], idx_v)
    for s in range(CHUNK // SUB):
        off = s * SUB
        cp = pltpu.async_copy(q_hbm.at[idx_v.at[pl.ds(off, SUB)]], q_v, sem)
        pltpu.sync_copy(x_hbm.at[pl.ds(base + off, SUB)], x_v)
        cp.wait()

        def body(j, _):
            acc = jnp.zeros((L,), jnp.float32)
            for k in range(HID // L):
                acc += x_v[j, pl.ds(k * L, L)] * q_v[j, pl.ds(k * L, L)]
            e_j = jnp.sum(acc)
            e_v[off + j] = e_j
            g = idx_v[off + j]
            m_v[g] = jnp.maximum(m_v[g], e_j)
            return 0

        lax.fori_loop(0, SUB, body, 0)
    pltpu.sync_copy(e_v, e_hbm.at[pl.ds(base, CHUNK)])
    pltpu.sync_copy(m_v, mpart_hbm.at[wid])


def kernel(node_embeddings, batch_indices, W_ih, W_hh, b_ih, b_hh, W_mlp, b_mlp):
    bi = batch_indices.astype(jnp.int32)
    x_pad = jnp.zeros((NPAD, HID), jnp.float32).at[:N].set(node_embeddings)
    idx_pad = jnp.full((NPAD,), NG, jnp.int32).at[:N].set(bi)

    h = jnp.zeros((NG, HID), jnp.float32)
    c = jnp.zeros((NG, HID), jnp.float32)
    r = jnp.zeros((NG, HID), jnp.float32)
    for _ in range(STEPS):
        q = h
        q_pad = jnp.zeros((GPAD, HID), jnp.float32).at[:NG].set(q)
        e_pad, mpart = _p1(x_pad, idx_pad, q_pad)
        e = e_pad[:N]
        m = jnp.max(mpart, axis=0)[:NG]
        m = jnp.where(jnp.isfinite(m), m, 0.0)
        a = jnp.exp(e - m[bi])
        denom = jax.ops.segment_sum(a, bi, num_segments=NG)
        a = a / jnp.maximum(denom[bi], 1e-12)
        r = jax.ops.segment_sum(a[:, None] * node_embeddings, bi, num_segments=NG)
        inp = jnp.concatenate([q, r], axis=1)
        gates = inp @ W_ih.T + h @ W_hh.T + b_ih + b_hh
        i_, f_, g_, o_ = jnp.split(gates, 4, axis=1)
        c = jax.nn.sigmoid(f_) * c + jax.nn.sigmoid(i_) * jnp.tanh(g_)
        h = jax.nn.sigmoid(o_) * jnp.tanh(c)
    out = jnp.concatenate([h, r], axis=1) @ W_mlp.T + b_mlp
    return out


# speedup vs baseline: 4.8991x; 4.8991x over previous
import functools

import jax
import jax.numpy as jnp
from jax import lax
from jax.experimental import pallas as pl
from jax.experimental.pallas import tpu as pltpu
from jax.experimental.pallas import tpu_sc as plsc

HID = 512
NG = 512
N = 10000
NC, NS, L = 2, 16, 16
NW = NC * NS
GPT = NG // NW
NPAD = 10240
NVR = NPAD // L
KH = HID // L
CH = 128
XG = CH // L
STEPS = 6

_mesh = plsc.VectorSubcoreMesh(core_axis_name="c", subcore_axis_name="s")
_sc_params = pltpu.CompilerParams(needs_layout_passes=False)


@functools.partial(
    pl.kernel,
    out_type=jax.ShapeDtypeStruct((NW, L), jnp.int32),
    mesh=_mesh,
    compiler_params=_sc_params,
    scratch_types=[
        pltpu.VMEM((NPAD,), jnp.int32),
        pltpu.VMEM((L,), jnp.int32),
    ],
)
def _spans(idx_hbm, out_hbm, idx_full, sbuf):
    wid = lax.axis_index("s") * NC + lax.axis_index("c")
    g_lo = wid * GPT
    lanes = lax.iota(jnp.int32, L)
    pltpu.sync_copy(idx_hbm, idx_full)

    def cnt_body(v, carry):
        lo_c, hi_c = carry
        ids = idx_full[pl.ds(v * L, L)]
        lo_c = lo_c + plsc.all_reduce_population_count(ids < g_lo)
        hi_c = hi_c + plsc.all_reduce_population_count(ids < g_lo + GPT)
        return lo_c, hi_c

    zi = jnp.zeros((L,), jnp.int32)
    cnt_lo, cnt_hi = lax.fori_loop(0, NVR, cnt_body, (zi, zi))
    sbuf[...] = jnp.where(lanes < 8, cnt_lo, cnt_hi)
    pltpu.sync_copy(sbuf, out_hbm.at[wid])


@functools.partial(
    pl.kernel,
    out_type=(
        jax.ShapeDtypeStruct((NPAD,), jnp.float32),
        jax.ShapeDtypeStruct((NW, L), jnp.float32),
    ),
    mesh=_mesh,
    compiler_params=_sc_params,
    scratch_types=[
        pltpu.VMEM((NPAD,), jnp.int32),
        pltpu.VMEM((2 * GPT, HID), jnp.float32),
        pltpu.VMEM((L, HID), jnp.float32),
        pltpu.VMEM((L, HID), jnp.float32),
        pltpu.VMEM((L,), jnp.int32),
        pltpu.VMEM((NPAD,), jnp.float32),
        pltpu.VMEM((L, L), jnp.float32),
        pltpu.VMEM((L,), jnp.float32),
        pltpu.VMEM((L,), jnp.float32),
        pltpu.SemaphoreType.DMA,
        pltpu.SemaphoreType.DMA,
    ],
)
def _attend(x_hbm, idx_hbm, q_hbm, spans_hbm, e_hbm, m_hbm, idx_full, q_loc,
            xa, xb, sbuf, e_loc, eparts, m_loc, zbuf, sema, semb):
    wid = lax.axis_index("s") * NC + lax.axis_index("c")
    g_lo = wid * GPT
    lanes = lax.iota(jnp.int32, L)
    m_loc[...] = jnp.full((L,), -jnp.inf, jnp.float32)
    zbuf[...] = jnp.zeros((L,), jnp.float32)

    pltpu.sync_copy(idx_hbm, idx_full)
    pltpu.sync_copy(q_hbm.at[pl.ds(g_lo, 2 * GPT)], q_loc)
    pltpu.sync_copy(spans_hbm.at[wid], sbuf)
    sv = sbuf[...]
    start = sv[0]
    end = sv[8]
    grp0 = start // L
    grp1 = (end + L - 1) // L
    npairs = jnp.maximum((grp1 - grp0 + 1) // 2, 1)

    def xrow(g):
        return jnp.clip(g, 0, NVR - 1) * L

    def prefetch(g, buf, sem):
        pltpu.async_copy(x_hbm.at[pl.ds(xrow(g), L)], buf, sem)

    def drain(buf, sem):
        pltpu.make_async_copy(x_hbm.at[pl.ds(0, L)], buf, sem).wait()

    def p1_grp(gi, xv):
        nb = gi * L
        ids16 = idx_full[pl.ds(nb, L)]
        g16 = ids16 - g_lo

        def dot4(uu, _):
            for vv in range(4):
                jj = uu * 4 + vv
                gs = plsc.load_gather(
                    idx_full, [jnp.full((L,), nb + jj, jnp.int32)]) - g_lo
                g_c = jnp.clip(gs, 0, 2 * GPT - 1)[0]
                acc = xv[jj, pl.ds(0, L)] * q_loc[g_c, pl.ds(0, L)]
                for k in range(1, KH):
                    acc += xv[jj, pl.ds(k * L, L)] * q_loc[g_c, pl.ds(k * L, L)]
                eparts[jj, :] = acc
            return 0

        lax.fori_loop(0, L // 4, dot4, 0)
        e16 = plsc.load_gather(eparts, [lanes, jnp.zeros((L,), jnp.int32)])
        for k in range(1, L):
            e16 += plsc.load_gather(eparts, [lanes, jnp.full((L,), k, jnp.int32)])
        e_loc[pl.ds((gi - grp0) * L, L)] = e16
        for jj in range(L):
            e_spl = jnp.full((L,), e16[jj], jnp.float32)
            gr_spl = jnp.full((L,), g16[jj], jnp.int32)
            v_spl = (gr_spl >= 0) & (gr_spl < GPT)
            g_spl = jnp.clip(gr_spl, 0, GPT - 1)
            mcur = plsc.load_gather(m_loc, [g_spl])
            mnew = jnp.where(v_spl, jnp.maximum(mcur, e_spl), mcur)
            plsc.store_scatter(m_loc, [g_spl], mnew)

    prefetch(grp0, xa, sema)
    prefetch(grp0 + 1, xb, semb)

    def p1_pair(i, _):
        gi = grp0 + 2 * i
        drain(xa, sema)
        p1_grp(gi, xa)
        prefetch(gi + 2, xa, sema)
        drain(xb, semb)
        p1_grp(gi + 1, xb)
        prefetch(gi + 3, xb, semb)
        return 0

    lax.fori_loop(0, npairs, p1_pair, 0)
    drain(xa, sema)
    drain(xb, semb)

    wlo = (start + L - 1) // L
    whi = (end + L - 1) // L

    def e_out(gi, _):
        pltpu.sync_copy(e_loc.at[pl.ds((gi - grp0) * L, L)],
                        e_hbm.at[pl.ds(gi * L, L)])
        return 0

    lax.fori_loop(wlo, whi, e_out, 0)

    @pl.when(wid == NW - 1)
    def _():
        def z_out(gi, _):
            pltpu.sync_copy(zbuf, e_hbm.at[pl.ds(gi * L, L)])
            return 0

        lax.fori_loop(whi, NVR, z_out, 0)

    mv = m_loc[...]
    m_loc[...] = jnp.where(mv > -3.0e38, mv, 0.0)
    pltpu.sync_copy(m_loc, m_hbm.at[wid])


NB = 2048
NCHUNK = NPAD // NB


def _rsum_body(ids_ref, e_ref, m_ref, x_ref, out_ref):
    gcol = lax.broadcasted_iota(jnp.int32, (NB, NG), 1)
    ids = ids_ref[0, :]
    b = (ids[:, None] == gcol).astype(jnp.float32)
    m_node = jnp.dot(b, m_ref[:].T, preferred_element_type=jnp.float32)
    s = jnp.exp(e_ref[0, :] - m_node[:, 0])
    sx = x_ref[:] * s[:, None]
    r_blk = jnp.dot(b.T, sx, preferred_element_type=jnp.float32)
    d_blk = jnp.dot(s[None, :], b, preferred_element_type=jnp.float32)

    @pl.when(pl.program_id(0) == 0)
    def _():
        out_ref[:] = jnp.zeros((NG + 8, HID), jnp.float32)

    out_ref[:NG, :] += r_blk
    out_ref[NG:NG + 1, :] += d_blk


_rsum = pl.pallas_call(
    _rsum_body,
    grid=(NCHUNK,),
    in_specs=[
        pl.BlockSpec((1, NB), lambda i: (0, i)),
        pl.BlockSpec((1, NB), lambda i: (0, i)),
        pl.BlockSpec((1, NG), lambda i: (0, 0)),
        pl.BlockSpec((NB, HID), lambda i: (i, 0)),
    ],
    out_specs=pl.BlockSpec((NG + 8, HID), lambda i: (0, 0)),
    out_shape=jax.ShapeDtypeStruct((NG + 8, HID), jnp.float32),
)


def _lstm_body(nd_ref, h_ref, c_ref, wq_ref, wr_ref, b_ref, hp_out, c_out,
               r_out):
    h = h_ref[:]
    d = jnp.maximum(nd_ref[NG, :], 1e-12)
    r = nd_ref[:NG, :] / d[:, None]
    gates = (
        jnp.dot(h, wq_ref[:], preferred_element_type=jnp.float32)
        + jnp.dot(r, wr_ref[:], preferred_element_type=jnp.float32)
        + b_ref[:]
    )
    i_ = gates[:, :HID]
    f_ = gates[:, HID:2 * HID]
    g_ = gates[:, 2 * HID:3 * HID]
    o_ = gates[:, 3 * HID:]
    sig_i = 1.0 / (1.0 + jnp.exp(-i_))
    sig_f = 1.0 / (1.0 + jnp.exp(-f_))
    sig_o = 1.0 / (1.0 + jnp.exp(-o_))
    c_new = sig_f * c_ref[:] + sig_i * jnp.tanh(g_)
    hp_out[:NG, :] = sig_o * jnp.tanh(c_new)
    hp_out[NG:, :] = jnp.zeros((GPT, HID), jnp.float32)
    c_out[:] = c_new
    r_out[:] = r


_lstm = pl.pallas_call(
    _lstm_body,
    out_shape=(
        jax.ShapeDtypeStruct((NG + GPT, HID), jnp.float32),
        jax.ShapeDtypeStruct((NG, HID), jnp.float32),
        jax.ShapeDtypeStruct((NG, HID), jnp.float32),
    ),
)


def _mlp_body(h_ref, r_ref, wh_ref, wr_ref, b_ref, out_ref):
    out_ref[:] = (
        jnp.dot(h_ref[:], wh_ref[:], preferred_element_type=jnp.float32)
        + jnp.dot(r_ref[:], wr_ref[:], preferred_element_type=jnp.float32)
        + b_ref[:]
    )


_mlp = pl.pallas_call(
    _mlp_body,
    out_shape=jax.ShapeDtypeStruct((NG, HID), jnp.float32),
)


def kernel(node_embeddings, batch_indices, W_ih, W_hh, b_ih, b_hh, W_mlp, b_mlp):
    bi = batch_indices.astype(jnp.int32)
    x_pad = jnp.zeros((NPAD, HID), jnp.float32).at[:N].set(node_embeddings)
    idx_pad = jnp.full((NPAD,), NG, jnp.int32).at[:N].set(bi)

    wq = W_ih[:, :HID].T + W_hh.T
    wr = W_ih[:, HID:].T
    b = (b_ih + b_hh)[None, :]
    wmh = W_mlp[:, :HID].T
    wmr = W_mlp[:, HID:].T

    hp = jnp.zeros((NG + GPT, HID), jnp.float32)
    c = jnp.zeros((NG, HID), jnp.float32)
    ids2 = idx_pad.reshape(1, NPAD)
    spans = _spans(idx_pad)
    for _ in range(STEPS):
        e, mpart = _attend(x_pad, idx_pad, hp, spans)
        nd = _rsum(ids2, e.reshape(1, NPAD), mpart.reshape(1, NG), x_pad)
        hp, c, r = _lstm(nd, hp[:NG], c, wq, wr, b)
    return _mlp(hp[:NG], r, wmh, wmr, b_mlp[None, :])
